# Initial kernel scaffold; baseline (speedup 1.0000x reference)
import jax
import jax.numpy as jnp
from jax.experimental import pallas as pl


def kernel(x, edge_index, W1, b1, W2, b2):
    raise NotImplementedError("write your pallas kernel here")


# speedup vs baseline: 7.3817x; 7.3817x over previous
import functools

import jax
import jax.numpy as jnp
from jax import lax
from jax.experimental import pallas as pl
from jax.experimental.pallas import tpu as pltpu
from jax.experimental.pallas import tpu_sc as plsc

N = 10000
D = 128
E = 320000

NC = 2
NS = 16
NW = NC * NS

CHUNK = 128
EPT = 10240
E_PAD = NW * EPT
NCHUNK = EPT // CHUNK

N_PAD = 10240
STRIPE = N_PAD // NS
DUMMY = N

_mesh = plsc.VectorSubcoreMesh(
    core_axis_name="c", subcore_axis_name="s", num_cores=NC, num_subcores=NS)


@functools.partial(
    pl.kernel,
    out_type=jax.ShapeDtypeStruct((NC * N_PAD, 16), jnp.float32),
    mesh=_mesh,
    scratch_types=[
        pltpu.VMEM((CHUNK,), jnp.int32),
        pltpu.VMEM((CHUNK, 16), jnp.float32),
        pltpu.VMEM((CHUNK, 16), jnp.float32),
        pltpu.VMEM_SHARED((N_PAD, 16), jnp.float32),
    ],
)
def _deg_kernel(dst_hbm, out_hbm, didx, ones, zbuf, accum):
    c = lax.axis_index("c")
    s = lax.axis_index("s")
    wid = s * NC + c
    base = wid * EPT

    one16 = jnp.full((16,), 1.0, jnp.float32)
    zero16 = jnp.zeros((16,), jnp.float32)

    def _fill(i, _):
        ones[i, :] = one16
        zbuf[i, :] = zero16
        return 0

    lax.fori_loop(0, CHUNK, _fill, 0)

    for k in range(STRIPE // CHUNK):
        pltpu.sync_copy(zbuf, accum.at[pl.ds(s * STRIPE + k * CHUNK, CHUNK)])
    plsc.subcore_barrier()

    def _body(j, _):
        pltpu.sync_copy(dst_hbm.at[pl.ds(base + j * CHUNK, CHUNK)], didx)
        pltpu.sync_copy(ones, accum.at[didx], add=True)
        return 0

    lax.fori_loop(0, NCHUNK, _body, 0)
    plsc.subcore_barrier()

    for k in range(STRIPE // CHUNK):
        r = s * STRIPE + k * CHUNK
        pltpu.sync_copy(accum.at[pl.ds(r, CHUNK)], zbuf)
        pltpu.sync_copy(zbuf, out_hbm.at[pl.ds(c * N_PAD + r, CHUNK)])


@functools.partial(
    pl.kernel,
    out_type=jax.ShapeDtypeStruct((NC * N_PAD, D), jnp.float32),
    mesh=_mesh,
    scratch_types=[
        pltpu.VMEM((CHUNK,), jnp.int32),
        pltpu.VMEM((CHUNK,), jnp.int32),
        pltpu.VMEM((CHUNK, D), jnp.float32),
        pltpu.VMEM_SHARED((N_PAD, D), jnp.float32),
        pltpu.SemaphoreType.DMA,
    ],
)
def _edge_kernel(g_hbm, src_hbm, dst_hbm, out_hbm, sidx, didx, rows, accum, sem):
    c = lax.axis_index("c")
    s = lax.axis_index("s")
    wid = s * NC + c
    base = wid * EPT

    zero16 = jnp.zeros((16,), jnp.float32)

    def _fill(i, _):
        for j in range(D // 16):
            rows[i, pl.ds(j * 16, 16)] = zero16
        return 0

    lax.fori_loop(0, CHUNK, _fill, 0)

    for k in range(STRIPE // CHUNK):
        pltpu.sync_copy(rows, accum.at[pl.ds(s * STRIPE + k * CHUNK, CHUNK)])
    plsc.subcore_barrier()

    def _body(j, _):
        off = base + j * CHUNK
        pltpu.sync_copy(src_hbm.at[pl.ds(off, CHUNK)], sidx)
        pltpu.async_copy(g_hbm.at[sidx], rows, sem).wait()
        pltpu.sync_copy(dst_hbm.at[pl.ds(off, CHUNK)], didx)
        pltpu.sync_copy(rows, accum.at[didx], add=True)
        return 0

    lax.fori_loop(0, NCHUNK, _body, 0)
    plsc.subcore_barrier()

    for k in range(STRIPE // CHUNK):
        r = s * STRIPE + k * CHUNK
        pltpu.sync_copy(accum.at[pl.ds(r, CHUNK)], rows)
        pltpu.sync_copy(rows, out_hbm.at[pl.ds(c * N_PAD + r, CHUNK)])


BLK = 1000
GRID = N // BLK


def _tc1_body(x_ref, w1_ref, d0_ref, d1_ref, g1_ref, dinv_ref):
    deg = 1.0 + d0_ref[...] + d1_ref[...]
    dinv = lax.rsqrt(deg)
    h = lax.dot_general(x_ref[...], w1_ref[...], (((1,), (1,)), ((), ())),
                        preferred_element_type=jnp.float32)
    g1_ref[...] = h * dinv
    dinv_ref[...] = dinv


def _tc1(x, w1, d0, d1):
    return pl.pallas_call(
        _tc1_body,
        grid=(GRID,),
        in_specs=[
            pl.BlockSpec((BLK, D), lambda i: (i, 0)),
            pl.BlockSpec((D, D), lambda i: (0, 0)),
            pl.BlockSpec((BLK, 1), lambda i: (i, 0)),
            pl.BlockSpec((BLK, 1), lambda i: (i, 0)),
        ],
        out_specs=[
            pl.BlockSpec((BLK, D), lambda i: (i, 0)),
            pl.BlockSpec((BLK, 1), lambda i: (i, 0)),
        ],
        out_shape=[
            jax.ShapeDtypeStruct((N, D), jnp.float32),
            jax.ShapeDtypeStruct((N, 1), jnp.float32),
        ],
    )(x, w1, d0, d1)


def _tc2_body(p_ref, g1_ref, dinv_ref, b1_ref, w2_ref, g2_ref):
    p = p_ref[0] + p_ref[1]
    dinv = dinv_ref[...]
    pre = dinv * (p + g1_ref[...]) + b1_ref[...]
    h1 = jnp.where(pre >= 0, pre, 0.01 * pre)
    g2_ref[...] = lax.dot_general(h1, w2_ref[...], (((1,), (1,)), ((), ())),
                                  preferred_element_type=jnp.float32) * dinv


def _tc2(parts, g1, dinv, b1, w2):
    return pl.pallas_call(
        _tc2_body,
        grid=(GRID,),
        in_specs=[
            pl.BlockSpec((NC, BLK, D), lambda i: (0, i, 0)),
            pl.BlockSpec((BLK, D), lambda i: (i, 0)),
            pl.BlockSpec((BLK, 1), lambda i: (i, 0)),
            pl.BlockSpec((1, D), lambda i: (0, 0)),
            pl.BlockSpec((D, D), lambda i: (0, 0)),
        ],
        out_specs=pl.BlockSpec((BLK, D), lambda i: (i, 0)),
        out_shape=jax.ShapeDtypeStruct((N, D), jnp.float32),
    )(parts, g1, dinv, b1, w2)


def _tc3_body(q_ref, g2_ref, dinv_ref, b2_ref, o_ref):
    q = q_ref[0] + q_ref[1]
    pre = dinv_ref[...] * (q + g2_ref[...]) + b2_ref[...]
    o_ref[...] = jnp.where(pre >= 0, pre, 0.01 * pre)


def _tc3(parts, g2, dinv, b2):
    return pl.pallas_call(
        _tc3_body,
        grid=(GRID,),
        in_specs=[
            pl.BlockSpec((NC, BLK, D), lambda i: (0, i, 0)),
            pl.BlockSpec((BLK, D), lambda i: (i, 0)),
            pl.BlockSpec((BLK, 1), lambda i: (i, 0)),
            pl.BlockSpec((1, D), lambda i: (0, 0)),
        ],
        out_specs=pl.BlockSpec((BLK, D), lambda i: (i, 0)),
        out_shape=jax.ShapeDtypeStruct((N, D), jnp.float32),
    )(parts, g2, dinv, b2)


def kernel(x, edge_index, W1, b1, W2, b2):
    src = edge_index[0]
    dst = edge_index[1]
    pad = E_PAD - E
    src_p = jnp.concatenate([src, jnp.zeros((pad,), jnp.int32)])
    dst_p = jnp.concatenate([dst, jnp.full((pad,), DUMMY, jnp.int32)])

    degp = _deg_kernel(dst_p).reshape(NC, N_PAD, 16)
    d0 = degp[0, :N, :1]
    d1 = degp[1, :N, :1]

    g1, dinv = _tc1(x, W1, d0, d1)
    p1 = _edge_kernel(g1, src_p, dst_p).reshape(NC, N_PAD, D)
    g2 = _tc2(p1, g1, dinv, b1.reshape(1, D), W2)
    p2 = _edge_kernel(g2, src_p, dst_p).reshape(NC, N_PAD, D)
    return _tc3(p2, g2, dinv, b2.reshape(1, D))


# speedup vs baseline: 9.0158x; 1.2214x over previous
import functools

import jax
import jax.numpy as jnp
from jax import lax
from jax.experimental import pallas as pl
from jax.experimental.pallas import tpu as pltpu
from jax.experimental.pallas import tpu_sc as plsc

N = 10000
D = 128
E = 320000

NC = 2
NS = 16
NW = NC * NS

CHUNK = 128
EPT = 10240
E_PAD = NW * EPT
NCHUNK = EPT // CHUNK

N_PAD = 10240
STRIPE = N_PAD // NS
DUMMY = N

_mesh = plsc.VectorSubcoreMesh(
    core_axis_name="c", subcore_axis_name="s", num_cores=NC, num_subcores=NS)


@functools.partial(
    pl.kernel,
    out_type=jax.ShapeDtypeStruct((NC * N_PAD, 16), jnp.float32),
    mesh=_mesh,
    scratch_types=[
        pltpu.VMEM((CHUNK,), jnp.int32),
        pltpu.VMEM((CHUNK, 16), jnp.float32),
        pltpu.VMEM((CHUNK, 16), jnp.float32),
        pltpu.VMEM_SHARED((N_PAD, 16), jnp.float32),
    ],
)
def _deg_kernel(dst_hbm, out_hbm, didx, ones, zbuf, accum):
    c = lax.axis_index("c")
    s = lax.axis_index("s")
    wid = s * NC + c
    base = wid * EPT

    one16 = jnp.full((16,), 1.0, jnp.float32)
    zero16 = jnp.zeros((16,), jnp.float32)

    def _fill(i, _):
        ones[i, :] = one16
        zbuf[i, :] = zero16
        return 0

    lax.fori_loop(0, CHUNK, _fill, 0)

    for k in range(STRIPE // CHUNK):
        pltpu.sync_copy(zbuf, accum.at[pl.ds(s * STRIPE + k * CHUNK, CHUNK)])
    plsc.subcore_barrier()

    def _body(j, _):
        pltpu.sync_copy(dst_hbm.at[pl.ds(base + j * CHUNK, CHUNK)], didx)
        pltpu.sync_copy(ones, accum.at[didx], add=True)
        return 0

    lax.fori_loop(0, NCHUNK, _body, 0)
    plsc.subcore_barrier()

    for k in range(STRIPE // CHUNK):
        r = s * STRIPE + k * CHUNK
        pltpu.sync_copy(accum.at[pl.ds(r, CHUNK)], zbuf)
        pltpu.sync_copy(zbuf, out_hbm.at[pl.ds(c * N_PAD + r, CHUNK)])


@functools.partial(
    pl.kernel,
    out_type=jax.ShapeDtypeStruct((NC * N_PAD, D), jnp.float32),
    mesh=_mesh,
    scratch_types=[
        pltpu.VMEM((EPT,), jnp.int32),
        pltpu.VMEM((CHUNK,), jnp.int32),
        pltpu.VMEM((CHUNK, D), jnp.float32),
        pltpu.VMEM((CHUNK, D), jnp.float32),
        pltpu.VMEM_SHARED((N_PAD, D), jnp.float32),
        pltpu.SemaphoreType.DMA,
        pltpu.SemaphoreType.DMA,
    ],
)
def _edge_kernel(g_hbm, src_hbm, dst_hbm, out_hbm, sflat, didx,
                 rows0, rows1, accum, sem0, sem1):
    c = lax.axis_index("c")
    s = lax.axis_index("s")
    wid = s * NC + c
    base = wid * EPT

    pltpu.sync_copy(src_hbm.at[pl.ds(base, EPT)], sflat)

    zero16 = jnp.zeros((16,), jnp.float32)

    def _fill(i, _):
        for j in range(D // 16):
            rows0[i, pl.ds(j * 16, 16)] = zero16
        return 0

    lax.fori_loop(0, CHUNK, _fill, 0)

    for k in range(STRIPE // CHUNK):
        pltpu.sync_copy(rows0, accum.at[pl.ds(s * STRIPE + k * CHUNK, CHUNK)])
    plsc.subcore_barrier()

    def _gidx(j):
        return sflat.at[pl.ds(j * CHUNK, CHUNK)]

    def _load_didx(j):
        pltpu.sync_copy(dst_hbm.at[pl.ds(base + j * CHUNK, CHUNK)], didx)

    pltpu.async_copy(g_hbm.at[_gidx(0)], rows0, sem0)

    def _body(j, _):
        c0 = 2 * j
        pltpu.async_copy(g_hbm.at[_gidx(c0 + 1)], rows1, sem1)
        pltpu.make_async_copy(g_hbm.at[_gidx(c0)], rows0, sem0).wait()
        _load_didx(c0)
        pltpu.sync_copy(rows0, accum.at[didx], add=True)
        nxt = jnp.minimum(c0 + 2, NCHUNK - 1)
        pltpu.async_copy(g_hbm.at[_gidx(nxt)], rows0, sem0)
        pltpu.make_async_copy(g_hbm.at[_gidx(c0 + 1)], rows1, sem1).wait()
        _load_didx(c0 + 1)
        pltpu.sync_copy(rows1, accum.at[didx], add=True)
        return 0

    lax.fori_loop(0, NCHUNK // 2, _body, 0)
    pltpu.make_async_copy(g_hbm.at[_gidx(NCHUNK - 1)], rows0, sem0).wait()
    plsc.subcore_barrier()

    for k in range(STRIPE // CHUNK):
        r = s * STRIPE + k * CHUNK
        pltpu.sync_copy(accum.at[pl.ds(r, CHUNK)], rows0)
        pltpu.sync_copy(rows0, out_hbm.at[pl.ds(c * N_PAD + r, CHUNK)])


BLK = 1000
GRID = N // BLK


def _tc1_body(x_ref, w1_ref, d0_ref, d1_ref, g1_ref, dinv_ref):
    deg = 1.0 + d0_ref[...] + d1_ref[...]
    dinv = lax.rsqrt(deg)
    h = lax.dot_general(x_ref[...], w1_ref[...], (((1,), (1,)), ((), ())),
                        preferred_element_type=jnp.float32)
    g1_ref[...] = h * dinv
    dinv_ref[...] = dinv


def _tc1(x, w1, d0, d1):
    return pl.pallas_call(
        _tc1_body,
        grid=(GRID,),
        in_specs=[
            pl.BlockSpec((BLK, D), lambda i: (i, 0)),
            pl.BlockSpec((D, D), lambda i: (0, 0)),
            pl.BlockSpec((BLK, 1), lambda i: (i, 0)),
            pl.BlockSpec((BLK, 1), lambda i: (i, 0)),
        ],
        out_specs=[
            pl.BlockSpec((BLK, D), lambda i: (i, 0)),
            pl.BlockSpec((BLK, 1), lambda i: (i, 0)),
        ],
        out_shape=[
            jax.ShapeDtypeStruct((N, D), jnp.float32),
            jax.ShapeDtypeStruct((N, 1), jnp.float32),
        ],
    )(x, w1, d0, d1)


def _tc2_body(p_ref, g1_ref, dinv_ref, b1_ref, w2_ref, g2_ref):
    p = p_ref[0] + p_ref[1]
    dinv = dinv_ref[...]
    pre = dinv * (p + g1_ref[...]) + b1_ref[...]
    h1 = jnp.where(pre >= 0, pre, 0.01 * pre)
    g2_ref[...] = lax.dot_general(h1, w2_ref[...], (((1,), (1,)), ((), ())),
                                  preferred_element_type=jnp.float32) * dinv


def _tc2(parts, g1, dinv, b1, w2):
    return pl.pallas_call(
        _tc2_body,
        grid=(GRID,),
        in_specs=[
            pl.BlockSpec((NC, BLK, D), lambda i: (0, i, 0)),
            pl.BlockSpec((BLK, D), lambda i: (i, 0)),
            pl.BlockSpec((BLK, 1), lambda i: (i, 0)),
            pl.BlockSpec((1, D), lambda i: (0, 0)),
            pl.BlockSpec((D, D), lambda i: (0, 0)),
        ],
        out_specs=pl.BlockSpec((BLK, D), lambda i: (i, 0)),
        out_shape=jax.ShapeDtypeStruct((N, D), jnp.float32),
    )(parts, g1, dinv, b1, w2)


def _tc3_body(q_ref, g2_ref, dinv_ref, b2_ref, o_ref):
    q = q_ref[0] + q_ref[1]
    pre = dinv_ref[...] * (q + g2_ref[...]) + b2_ref[...]
    o_ref[...] = jnp.where(pre >= 0, pre, 0.01 * pre)


def _tc3(parts, g2, dinv, b2):
    return pl.pallas_call(
        _tc3_body,
        grid=(GRID,),
        in_specs=[
            pl.BlockSpec((NC, BLK, D), lambda i: (0, i, 0)),
            pl.BlockSpec((BLK, D), lambda i: (i, 0)),
            pl.BlockSpec((BLK, 1), lambda i: (i, 0)),
            pl.BlockSpec((1, D), lambda i: (0, 0)),
        ],
        out_specs=pl.BlockSpec((BLK, D), lambda i: (i, 0)),
        out_shape=jax.ShapeDtypeStruct((N, D), jnp.float32),
    )(parts, g2, dinv, b2)


def kernel(x, edge_index, W1, b1, W2, b2):
    src = edge_index[0]
    dst = edge_index[1]
    pad = E_PAD - E
    src_p = jnp.concatenate([src, jnp.zeros((pad,), jnp.int32)])
    dst_p = jnp.concatenate([dst, jnp.full((pad,), DUMMY, jnp.int32)])

    degp = _deg_kernel(dst_p).reshape(NC, N_PAD, 16)
    d0 = degp[0, :N, :1]
    d1 = degp[1, :N, :1]

    g1, dinv = _tc1(x, W1, d0, d1)
    p1 = _edge_kernel(g1, src_p, dst_p).reshape(NC, N_PAD, D)
    g2 = _tc2(p1, g1, dinv, b1.reshape(1, D), W2)
    p2 = _edge_kernel(g2, src_p, dst_p).reshape(NC, N_PAD, D)
    return _tc3(p2, g2, dinv, b2.reshape(1, D))
